# baseline (speedup 1.0000x reference)
import functools

import jax
import jax.numpy as jnp
from jax import lax
from jax.experimental import pallas as pl
from jax.experimental.pallas import tpu as pltpu
from jax.experimental.pallas import tpu_sc as plsc

_N = 10000
_E = 160000
_H = 384
_G = 128
_NC = 2
_NS = 16
_CH = 128
_NCHUNK = 80
_EPT = _CH * _NCHUNK
_EPAD = _EPT * _NS
_HALFN = 5200
_NACCH = 5248
_RPT = _NACCH // _NS

_NBUF = 4

_BN = 400
_BE = 2000

_F32 = jnp.float32


def _dot(a, b):
    return jnp.dot(a, b, preferred_element_type=_F32)


@functools.lru_cache(maxsize=None)
def _make_seg_sum(rows: int, with_cnt: bool):
    del rows
    n_out = 8 if with_cnt else 6
    mesh = plsc.VectorSubcoreMesh(core_axis_name="c", subcore_axis_name="s")

    @functools.partial(
        pl.kernel,
        out_type=jax.ShapeDtypeStruct((n_out, _NACCH, _G), _F32),
        mesh=mesh,
        scratch_types=[
            pltpu.VMEM((_NCHUNK, _CH), jnp.int32),
            pltpu.VMEM((_NCHUNK, _CH), jnp.int32),
            [pltpu.VMEM((_CH, _G), _F32) for _ in range(_NBUF)],
            pltpu.VMEM_SHARED((_NACCH, _G), _F32),
            [pltpu.SemaphoreType.DMA for _ in range(_NBUF)],
            [pltpu.SemaphoreType.DMA for _ in range(_NBUF)],
        ],
    )
    def seg_sum(t0_hbm, t1_hbm, t2_hbm, src_hbm, dst_hbm, z_hbm, ones_hbm,
                out_hbm, src_v, dst_v, bufs, acc, gsem, ssem):
        c = lax.axis_index("c")
        s = lax.axis_index("s")
        row0 = s * _RPT
        rows_sl = pl.ds(row0, _RPT)
        pltpu.sync_copy(src_hbm.at[s], src_v)
        pltpu.sync_copy(dst_hbm.at[s], dst_v)
        base = c * _HALFN

        def remap(j, carry):
            for k in range(_CH // 16):
                lv = dst_v[j, pl.ds(16 * k, 16)] - base
                ok = (lv >= 0) & (lv < _HALFN)
                dst_v[j, pl.ds(16 * k, 16)] = jnp.where(ok, lv, _NACCH - 1)
            return carry

        lax.fori_loop(0, _NCHUNK, remap, 0)

        def start_zero():
            pltpu.sync_copy(z_hbm.at[rows_sl], acc.at[rows_sl])
            plsc.subcore_barrier()

        def finish(out_slot):
            plsc.subcore_barrier()
            pltpu.sync_copy(acc.at[rows_sl], out_hbm.at[out_slot].at[rows_sl])
            plsc.subcore_barrier()

        def run(table, out_slot):
            start_zero()
            for b in range(_NBUF):
                pltpu.async_copy(table.at[src_v.at[b]], bufs[b], gsem[b])

            def body(i, carry):
                j = _NBUF * i
                for b in range(_NBUF):
                    pltpu.make_async_copy(
                        table.at[src_v.at[j + b]], bufs[b], gsem[b]).wait()
                    pltpu.async_copy(
                        bufs[b], acc.at[dst_v.at[j + b]], ssem[b], add=True)
                for b in range(_NBUF):
                    pltpu.make_async_copy(
                        bufs[b], acc.at[dst_v.at[j + b]], ssem[b]).wait()

                    @pl.when(i < _NCHUNK // _NBUF - 1)
                    def _():
                        pltpu.async_copy(
                            table.at[src_v.at[j + _NBUF + b]], bufs[b],
                            gsem[b])

                return carry

            lax.fori_loop(0, _NCHUNK // _NBUF, body, 0)
            finish(out_slot)

        run(t0_hbm, c)
        run(t1_hbm, 2 + c)
        run(t2_hbm, 4 + c)

        if with_cnt:
            start_zero()
            pltpu.sync_copy(ones_hbm, bufs[0])

            def cbody(i, carry):
                pltpu.sync_copy(bufs[0], acc.at[dst_v.at[i]], add=True)
                return carry

            lax.fori_loop(0, _NCHUNK, cbody, 0)
            finish(6 + c)

    return seg_sum


def _full(w):
    return pl.BlockSpec(w.shape, lambda i: (0,) * w.ndim)


def _node_mlp(x, geo_w, geo_b, n1_w, n1_b, n2_w, n2_b):
    def body(x_ref, gw, gb, w1, b1, w2, b2, h0, h1, h2):
        xg = x_ref[:, :128]
        xo = x_ref[:, 128:]
        geo = jnp.maximum(_dot(xg, gw[...]) + gb[...], 0.0)
        hp = jnp.maximum(
            _dot(geo, w1[:16, :]) + _dot(xo, w1[16:, :]) + b1[...], 0.0)
        h = _dot(hp, w2[...]) + b2[...]
        h0[...] = h[:, :_G]
        h1[...] = h[:, _G:2 * _G]
        h2[...] = h[:, 2 * _G:]

    gspec = pl.BlockSpec((_BN, _G), lambda i: (i, 0))
    gshape = jax.ShapeDtypeStruct((_N, _G), _F32)
    return pl.pallas_call(
        body,
        grid=(_N // _BN,),
        in_specs=[
            pl.BlockSpec((_BN, 256), lambda i: (i, 0)),
            _full(geo_w), _full(geo_b), _full(n1_w), _full(n1_b),
            _full(n2_w), _full(n2_b),
        ],
        out_specs=[gspec, gspec, gspec],
        out_shape=[gshape, gshape, gshape],
    )(x, geo_w, geo_b, n1_w, n1_b, n2_w, n2_b)


def _edge_tables(edge_attr, e1_w, e1_b):
    def body(a_ref, w, b, t0, t1, t2):
        r = jnp.maximum(_dot(a_ref[...], w[...]) + b[...], 0.0)
        t0[...] = r[:, :_G]
        t1[...] = r[:, _G:2 * _G]
        t2[...] = r[:, 2 * _G:]

    gspec = pl.BlockSpec((_BE, _G), lambda i: (i, 0))
    gshape = jax.ShapeDtypeStruct((_E, _G), _F32)
    return pl.pallas_call(
        body,
        grid=(_E // _BE,),
        in_specs=[
            pl.BlockSpec((_BE, 16), lambda i: (i, 0)),
            _full(e1_w), _full(e1_b),
        ],
        out_specs=[gspec, gspec, gspec],
        out_shape=[gshape, gshape, gshape],
    )(edge_attr, e1_w, e1_b)


def _sspec(g):
    return pl.BlockSpec(
        (1, _BN, _G), lambda i, g=g: (2 * g + i // 13, i - 13 * (i // 13), 0))


def _ce_combine(se, e2_w, e2_b):
    def body(s0r, s1r, s2r, ctr, w, b, ce):
        a = (_dot(s0r[0], w[:_G, :]) + _dot(s1r[0], w[_G:2 * _G, :])
             + _dot(s2r[0], w[2 * _G:, :]))
        ce[...] = a + ctr[0][:, :1] * b[...]

    return pl.pallas_call(
        body,
        grid=(_N // _BN,),
        in_specs=[_sspec(0), _sspec(1), _sspec(2), _sspec(3),
                  _full(e2_w), _full(e2_b)],
        out_specs=pl.BlockSpec((_BN, _H), lambda i: (i, 0)),
        out_shape=jax.ShapeDtypeStruct((_N, _H), _F32),
    )(se, se, se, se, e2_w, e2_b)


def _layer(h0, h1, h2, sh, ce, wx, we, u1, ub1, u2, ub2, lg, lb):
    def body(h0r, h1r, h2r, s0r, s1r, s2r, cer, wxr, wer, u1r, ub1r,
             u2r, ub2r, lgr, lbr, o0, o1, o2, osum):
        i = pl.program_id(0)
        agg = (_dot(s0r[0], wxr[:_G, :]) + _dot(s1r[0], wxr[_G:2 * _G, :])
               + _dot(s2r[0], wxr[2 * _G:, :])
               + _dot(cer[...], wer[...]))
        t = jnp.maximum(_dot(agg, u1r[...]) + ub1r[...], 0.0)
        upd = _dot(t, u2r[...]) + ub2r[...]
        r = jnp.concatenate([h0r[...], h1r[...], h2r[...]], axis=1) + upd
        m = jnp.mean(r, axis=1, keepdims=True)
        d = r - m
        v = jnp.mean(d * d, axis=1, keepdims=True)
        hn = d * lax.rsqrt(v + 1e-5) * lgr[...] + lbr[...]
        o0[...] = hn[:, :_G]
        o1[...] = hn[:, _G:2 * _G]
        o2[...] = hn[:, 2 * _G:]

        @pl.when(i == 0)
        def _():
            osum[...] = jnp.zeros((1, _H), _F32)

        osum[...] += jnp.sum(hn, axis=0, keepdims=True)

    gspec = pl.BlockSpec((_BN, _G), lambda i: (i, 0))
    gshape = jax.ShapeDtypeStruct((_N, _G), _F32)
    return pl.pallas_call(
        body,
        grid=(_N // _BN,),
        in_specs=[
            gspec, gspec, gspec, _sspec(0), _sspec(1), _sspec(2),
            pl.BlockSpec((_BN, _H), lambda i: (i, 0)),
            _full(wx), _full(we), _full(u1), _full(ub1), _full(u2),
            _full(ub2), _full(lg), _full(lb),
        ],
        out_specs=[gspec, gspec, gspec,
                   pl.BlockSpec((1, _H), lambda i: (0, 0))],
        out_shape=[gshape, gshape, gshape,
                   jax.ShapeDtypeStruct((1, _H), _F32)],
    )(h0, h1, h2, sh, sh, sh, ce, wx, we, u1, ub1, u2, ub2, lg, lb)


def _head(hsum, h1_w, h1_b, h2_w, h2_b):
    def body(sr, w1, b1, w2, b2, o):
        g = sr[...] * (1.0 / _N)
        t = jnp.maximum(_dot(g, w1[...]) + b1[...], 0.0)
        o[...] = _dot(t, w2[...]) + b2[...]

    return pl.pallas_call(
        body,
        grid=(1,),
        in_specs=[_full(hsum), _full(h1_w), _full(h1_b), _full(h2_w),
                  _full(h2_b)],
        out_specs=pl.BlockSpec((1, 128), lambda i: (0, 0)),
        out_shape=jax.ShapeDtypeStruct((1, 128), _F32),
    )(hsum, h1_w, h1_b, h2_w, h2_b)


def kernel(x, edge_index, edge_attr, geo_w, geo_b, n1_w, n1_b, n2_w, n2_b,
           e1_w, e1_b, e2_w, e2_b, msgx_w, msge_w, upd1_w, upd1_b, upd2_w,
           upd2_b, ln_g, ln_b, h1_w, h1_b, h2_w, h2_b):
    src = edge_index[0]
    dst = edge_index[1]
    pad = _EPAD - _E
    srcp = jnp.concatenate(
        [src, jnp.zeros((pad,), jnp.int32)]).reshape(_NS, _NCHUNK, _CH)
    dstp = jnp.concatenate(
        [dst, jnp.full((pad,), _N, jnp.int32)]).reshape(_NS, _NCHUNK, _CH)
    eidp = jnp.concatenate(
        [jnp.arange(_E, dtype=jnp.int32),
         jnp.zeros((pad,), jnp.int32)]).reshape(_NS, _NCHUNK, _CH)
    zacc = jnp.zeros((_NACCH, _G), _F32)
    ones_row = jnp.zeros((_CH, _G), _F32).at[:, 0].set(1.0)

    r2 = lambda b: b.reshape(1, -1)

    h0, h1, h2 = _node_mlp(x, geo_w, r2(geo_b), n1_w, r2(n1_b),
                           n2_w, r2(n2_b))
    t0, t1, t2 = _edge_tables(edge_attr, e1_w, r2(e1_b))
    se = _make_seg_sum(_E, True)(t0, t1, t2, eidp, dstp, zacc, ones_row)
    ce = _ce_combine(se, e2_w, r2(e2_b))

    hsum = None
    for l in range(3):
        sh = _make_seg_sum(_N, False)(h0, h1, h2, srcp, dstp, zacc, ones_row)
        h0, h1, h2, hsum = _layer(h0, h1, h2, sh, ce,
                                  msgx_w[l], msge_w[l],
                                  upd1_w[l], r2(upd1_b[l]),
                                  upd2_w[l], r2(upd2_b[l]),
                                  r2(ln_g[l]), r2(ln_b[l]))

    return _head(hsum, h1_w, r2(h1_b), h2_w, r2(h2_b))


# speedup vs baseline: 1.0666x; 1.0666x over previous
import functools

import jax
import jax.numpy as jnp
from jax import lax
from jax.experimental import pallas as pl
from jax.experimental.pallas import tpu as pltpu
from jax.experimental.pallas import tpu_sc as plsc

_N = 10000
_E = 160000
_H = 384
_G = 128
_NC = 2
_NS = 16
_CH = 128
_NCHUNK = 80
_EPT = _CH * _NCHUNK
_EPAD = _EPT * _NS
_HALFN = 5200
_SPREAD = 128
_NACCH = 5376
_RPT = _NACCH // _NS

_NBUF = 4

_BN = 400
_BE = 2000

_F32 = jnp.float32


def _dot(a, b):
    return jnp.dot(a, b, preferred_element_type=_F32)


@functools.lru_cache(maxsize=None)
def _make_seg_sum(rows: int, with_cnt: bool):
    del rows
    n_out = 8 if with_cnt else 6
    mesh = plsc.VectorSubcoreMesh(core_axis_name="c", subcore_axis_name="s")

    @functools.partial(
        pl.kernel,
        out_type=jax.ShapeDtypeStruct((n_out, _NACCH, _G), _F32),
        mesh=mesh,
        scratch_types=[
            pltpu.VMEM((_NCHUNK, _CH), jnp.int32),
            pltpu.VMEM((_NCHUNK, _CH), jnp.int32),
            [pltpu.VMEM((_CH, _G), _F32) for _ in range(_NBUF)],
            pltpu.VMEM_SHARED((_NACCH, _G), _F32),
            [pltpu.SemaphoreType.DMA for _ in range(_NBUF)],
            [pltpu.SemaphoreType.DMA for _ in range(_NBUF)],
        ],
    )
    def seg_sum(t0_hbm, t1_hbm, t2_hbm, src_hbm, dst_hbm, z_hbm, ones_hbm,
                out_hbm, src_v, dst_v, bufs, acc, gsem, ssem):
        c = lax.axis_index("c")
        s = lax.axis_index("s")
        row0 = s * _RPT
        rows_sl = pl.ds(row0, _RPT)
        pltpu.sync_copy(src_hbm.at[s], src_v)
        pltpu.sync_copy(dst_hbm.at[s], dst_v)
        base = c * _HALFN

        def remap(j, carry):
            for k in range(_CH // 16):
                dummy = _HALFN + (
                    (16 * k + lax.iota(jnp.int32, 16)) & (_SPREAD - 1))
                lv = dst_v[j, pl.ds(16 * k, 16)] - base
                ok = (lv >= 0) & (lv < _HALFN)
                dst_v[j, pl.ds(16 * k, 16)] = jnp.where(ok, lv, dummy)
            return carry

        lax.fori_loop(0, _NCHUNK, remap, 0)

        def start_zero():
            pltpu.sync_copy(z_hbm.at[rows_sl], acc.at[rows_sl])
            plsc.subcore_barrier()

        def finish(out_slot):
            plsc.subcore_barrier()
            pltpu.sync_copy(acc.at[rows_sl], out_hbm.at[out_slot].at[rows_sl])
            plsc.subcore_barrier()

        def run(table, out_slot):
            start_zero()
            for b in range(_NBUF):
                pltpu.async_copy(table.at[src_v.at[b]], bufs[b], gsem[b])

            def body(i, carry):
                j = _NBUF * i
                for b in range(_NBUF):
                    pltpu.make_async_copy(
                        table.at[src_v.at[j + b]], bufs[b], gsem[b]).wait()
                    pltpu.async_copy(
                        bufs[b], acc.at[dst_v.at[j + b]], ssem[b], add=True)
                for b in range(_NBUF):
                    pltpu.make_async_copy(
                        bufs[b], acc.at[dst_v.at[j + b]], ssem[b]).wait()

                    @pl.when(i < _NCHUNK // _NBUF - 1)
                    def _():
                        pltpu.async_copy(
                            table.at[src_v.at[j + _NBUF + b]], bufs[b],
                            gsem[b])

                return carry

            lax.fori_loop(0, _NCHUNK // _NBUF, body, 0)
            finish(out_slot)

        run(t0_hbm, c)
        run(t1_hbm, 2 + c)
        run(t2_hbm, 4 + c)

        if with_cnt:
            start_zero()
            pltpu.sync_copy(ones_hbm, bufs[0])

            def cbody(i, carry):
                pltpu.sync_copy(bufs[0], acc.at[dst_v.at[i]], add=True)
                return carry

            lax.fori_loop(0, _NCHUNK, cbody, 0)
            finish(6 + c)

    return seg_sum


def _full(w):
    return pl.BlockSpec(w.shape, lambda i: (0,) * w.ndim)


def _node_mlp(x, geo_w, geo_b, n1_w, n1_b, n2_w, n2_b):
    def body(x_ref, gw, gb, w1, b1, w2, b2, h0, h1, h2):
        xg = x_ref[:, :128]
        xo = x_ref[:, 128:]
        geo = jnp.maximum(_dot(xg, gw[...]) + gb[...], 0.0)
        hp = jnp.maximum(
            _dot(geo, w1[:16, :]) + _dot(xo, w1[16:, :]) + b1[...], 0.0)
        h = _dot(hp, w2[...]) + b2[...]
        h0[...] = h[:, :_G]
        h1[...] = h[:, _G:2 * _G]
        h2[...] = h[:, 2 * _G:]

    gspec = pl.BlockSpec((_BN, _G), lambda i: (i, 0))
    gshape = jax.ShapeDtypeStruct((_N, _G), _F32)
    return pl.pallas_call(
        body,
        grid=(_N // _BN,),
        in_specs=[
            pl.BlockSpec((_BN, 256), lambda i: (i, 0)),
            _full(geo_w), _full(geo_b), _full(n1_w), _full(n1_b),
            _full(n2_w), _full(n2_b),
        ],
        out_specs=[gspec, gspec, gspec],
        out_shape=[gshape, gshape, gshape],
    )(x, geo_w, geo_b, n1_w, n1_b, n2_w, n2_b)


def _edge_tables(edge_attr, e1_w, e1_b):
    def body(a_ref, w, b, t0, t1, t2):
        r = jnp.maximum(_dot(a_ref[...], w[...]) + b[...], 0.0)
        t0[...] = r[:, :_G]
        t1[...] = r[:, _G:2 * _G]
        t2[...] = r[:, 2 * _G:]

    gspec = pl.BlockSpec((_BE, _G), lambda i: (i, 0))
    gshape = jax.ShapeDtypeStruct((_E, _G), _F32)
    return pl.pallas_call(
        body,
        grid=(_E // _BE,),
        in_specs=[
            pl.BlockSpec((_BE, 16), lambda i: (i, 0)),
            _full(e1_w), _full(e1_b),
        ],
        out_specs=[gspec, gspec, gspec],
        out_shape=[gshape, gshape, gshape],
    )(edge_attr, e1_w, e1_b)


def _sspec(g):
    return pl.BlockSpec(
        (1, _BN, _G), lambda i, g=g: (2 * g + i // 13, i - 13 * (i // 13), 0))


def _ce_combine(se, e2_w, e2_b):
    def body(s0r, s1r, s2r, ctr, w, b, ce):
        a = (_dot(s0r[0], w[:_G, :]) + _dot(s1r[0], w[_G:2 * _G, :])
             + _dot(s2r[0], w[2 * _G:, :]))
        ce[...] = a + ctr[0][:, :1] * b[...]

    return pl.pallas_call(
        body,
        grid=(_N // _BN,),
        in_specs=[_sspec(0), _sspec(1), _sspec(2), _sspec(3),
                  _full(e2_w), _full(e2_b)],
        out_specs=pl.BlockSpec((_BN, _H), lambda i: (i, 0)),
        out_shape=jax.ShapeDtypeStruct((_N, _H), _F32),
    )(se, se, se, se, e2_w, e2_b)


def _layer(h0, h1, h2, sh, ce, wx, we, u1, ub1, u2, ub2, lg, lb):
    def body(h0r, h1r, h2r, s0r, s1r, s2r, cer, wxr, wer, u1r, ub1r,
             u2r, ub2r, lgr, lbr, o0, o1, o2, osum):
        i = pl.program_id(0)
        agg = (_dot(s0r[0], wxr[:_G, :]) + _dot(s1r[0], wxr[_G:2 * _G, :])
               + _dot(s2r[0], wxr[2 * _G:, :])
               + _dot(cer[...], wer[...]))
        t = jnp.maximum(_dot(agg, u1r[...]) + ub1r[...], 0.0)
        upd = _dot(t, u2r[...]) + ub2r[...]
        r = jnp.concatenate([h0r[...], h1r[...], h2r[...]], axis=1) + upd
        m = jnp.mean(r, axis=1, keepdims=True)
        d = r - m
        v = jnp.mean(d * d, axis=1, keepdims=True)
        hn = d * lax.rsqrt(v + 1e-5) * lgr[...] + lbr[...]
        o0[...] = hn[:, :_G]
        o1[...] = hn[:, _G:2 * _G]
        o2[...] = hn[:, 2 * _G:]

        @pl.when(i == 0)
        def _():
            osum[...] = jnp.zeros((1, _H), _F32)

        osum[...] += jnp.sum(hn, axis=0, keepdims=True)

    gspec = pl.BlockSpec((_BN, _G), lambda i: (i, 0))
    gshape = jax.ShapeDtypeStruct((_N, _G), _F32)
    return pl.pallas_call(
        body,
        grid=(_N // _BN,),
        in_specs=[
            gspec, gspec, gspec, _sspec(0), _sspec(1), _sspec(2),
            pl.BlockSpec((_BN, _H), lambda i: (i, 0)),
            _full(wx), _full(we), _full(u1), _full(ub1), _full(u2),
            _full(ub2), _full(lg), _full(lb),
        ],
        out_specs=[gspec, gspec, gspec,
                   pl.BlockSpec((1, _H), lambda i: (0, 0))],
        out_shape=[gshape, gshape, gshape,
                   jax.ShapeDtypeStruct((1, _H), _F32)],
    )(h0, h1, h2, sh, sh, sh, ce, wx, we, u1, ub1, u2, ub2, lg, lb)


def _head(hsum, h1_w, h1_b, h2_w, h2_b):
    def body(sr, w1, b1, w2, b2, o):
        g = sr[...] * (1.0 / _N)
        t = jnp.maximum(_dot(g, w1[...]) + b1[...], 0.0)
        o[...] = _dot(t, w2[...]) + b2[...]

    return pl.pallas_call(
        body,
        grid=(1,),
        in_specs=[_full(hsum), _full(h1_w), _full(h1_b), _full(h2_w),
                  _full(h2_b)],
        out_specs=pl.BlockSpec((1, 128), lambda i: (0, 0)),
        out_shape=jax.ShapeDtypeStruct((1, 128), _F32),
    )(hsum, h1_w, h1_b, h2_w, h2_b)


def kernel(x, edge_index, edge_attr, geo_w, geo_b, n1_w, n1_b, n2_w, n2_b,
           e1_w, e1_b, e2_w, e2_b, msgx_w, msge_w, upd1_w, upd1_b, upd2_w,
           upd2_b, ln_g, ln_b, h1_w, h1_b, h2_w, h2_b):
    src = edge_index[0]
    dst = edge_index[1]
    pad = _EPAD - _E
    srcp = jnp.concatenate(
        [src, jnp.zeros((pad,), jnp.int32)]).reshape(_NS, _NCHUNK, _CH)
    dstp = jnp.concatenate(
        [dst, jnp.full((pad,), _N, jnp.int32)]).reshape(_NS, _NCHUNK, _CH)
    eidp = jnp.concatenate(
        [jnp.arange(_E, dtype=jnp.int32),
         jnp.zeros((pad,), jnp.int32)]).reshape(_NS, _NCHUNK, _CH)
    zacc = jnp.zeros((_NACCH, _G), _F32)
    ones_row = jnp.zeros((_CH, _G), _F32).at[:, 0].set(1.0)

    r2 = lambda b: b.reshape(1, -1)

    h0, h1, h2 = _node_mlp(x, geo_w, r2(geo_b), n1_w, r2(n1_b),
                           n2_w, r2(n2_b))
    t0, t1, t2 = _edge_tables(edge_attr, e1_w, r2(e1_b))
    se = _make_seg_sum(_E, True)(t0, t1, t2, eidp, dstp, zacc, ones_row)
    ce = _ce_combine(se, e2_w, r2(e2_b))

    hsum = None
    for l in range(3):
        sh = _make_seg_sum(_N, False)(h0, h1, h2, srcp, dstp, zacc, ones_row)
        h0, h1, h2, hsum = _layer(h0, h1, h2, sh, ce,
                                  msgx_w[l], msge_w[l],
                                  upd1_w[l], r2(upd1_b[l]),
                                  upd2_w[l], r2(upd2_b[l]),
                                  r2(ln_g[l]), r2(ln_b[l]))

    return _head(hsum, h1_w, r2(h1_b), h2_w, r2(h2_b))


# speedup vs baseline: 1.2766x; 1.1968x over previous
import functools

import jax
import jax.numpy as jnp
from jax import lax
from jax.experimental import pallas as pl
from jax.experimental.pallas import tpu as pltpu
from jax.experimental.pallas import tpu_sc as plsc

_N = 10000
_E = 160000
_H = 384
_G = 128
_NC = 2
_NS = 16
_CH = 128
_NCHUNK = 80
_EPT = _CH * _NCHUNK
_EPAD = _EPT * _NS
_HALFN = 5200
_SPREAD = 128
_NACCH = 5376
_RPT = _NACCH // _NS
_NBUF = 4

_BN = 400
_BE = 2048

_F32 = jnp.float32


def _dot(a, b):
    return jnp.dot(a, b, preferred_element_type=_F32)


@functools.lru_cache(maxsize=None)
def _make_seg_sum(rows: int, with_cnt: bool, linear: bool):
    del rows
    n_out = 8 if with_cnt else 6
    mesh = plsc.VectorSubcoreMesh(core_axis_name="c", subcore_axis_name="s")

    @functools.partial(
        pl.kernel,
        out_type=jax.ShapeDtypeStruct((n_out, _NACCH, _G), _F32),
        mesh=mesh,
        scratch_types=[
            pltpu.VMEM((_NCHUNK, _CH), jnp.int32),
            pltpu.VMEM((_NCHUNK, _CH), jnp.int32),
            [pltpu.VMEM((_CH, _G), _F32) for _ in range(_NBUF)],
            pltpu.VMEM_SHARED((_NACCH, _G), _F32),
            [pltpu.SemaphoreType.DMA for _ in range(_NBUF)],
            [pltpu.SemaphoreType.DMA for _ in range(_NBUF)],
        ],
    )
    def seg_sum(t0_hbm, t1_hbm, t2_hbm, src_hbm, dst_hbm, z_hbm, ones_hbm,
                out_hbm, src_v, dst_v, bufs, acc, gsem, ssem):
        c = lax.axis_index("c")
        s = lax.axis_index("s")
        row0 = s * _RPT
        rows_sl = pl.ds(row0, _RPT)
        if not linear:
            pltpu.sync_copy(src_hbm.at[s], src_v)
        pltpu.sync_copy(dst_hbm.at[s], dst_v)
        base = c * _HALFN
        trow0 = s * _EPT

        def remap(j, carry):
            for k in range(_CH // 16):
                dummy = _HALFN + (
                    (16 * k + lax.iota(jnp.int32, 16)) & (_SPREAD - 1))
                lv = dst_v[j, pl.ds(16 * k, 16)] - base
                ok = (lv >= 0) & (lv < _HALFN)
                dst_v[j, pl.ds(16 * k, 16)] = jnp.where(ok, lv, dummy)
            return carry

        lax.fori_loop(0, _NCHUNK, remap, 0)

        def start_zero():
            pltpu.sync_copy(z_hbm.at[rows_sl], acc.at[rows_sl])
            plsc.subcore_barrier()

        def finish(out_slot):
            plsc.subcore_barrier()
            pltpu.sync_copy(acc.at[rows_sl], out_hbm.at[out_slot].at[rows_sl])
            plsc.subcore_barrier()

        def run(table, out_slot):
            def src_of(j):
                if linear:
                    return table.at[pl.ds(trow0 + _CH * j, _CH)]
                return table.at[src_v.at[j]]

            start_zero()
            for b in range(_NBUF):
                pltpu.async_copy(src_of(b), bufs[b], gsem[b])

            def body(i, carry):
                j = _NBUF * i
                for b in range(_NBUF):
                    pltpu.make_async_copy(
                        src_of(j + b), bufs[b], gsem[b]).wait()
                    pltpu.async_copy(
                        bufs[b], acc.at[dst_v.at[j + b]], ssem[b], add=True)
                for b in range(_NBUF):
                    pltpu.make_async_copy(
                        bufs[b], acc.at[dst_v.at[j + b]], ssem[b]).wait()

                    @pl.when(i < _NCHUNK // _NBUF - 1)
                    def _():
                        pltpu.async_copy(
                            src_of(j + _NBUF + b), bufs[b], gsem[b])

                return carry

            lax.fori_loop(0, _NCHUNK // _NBUF, body, 0)
            finish(out_slot)

        run(t0_hbm, c)
        run(t1_hbm, 2 + c)
        run(t2_hbm, 4 + c)

        if with_cnt:
            start_zero()
            pltpu.sync_copy(ones_hbm, bufs[0])

            def cbody(i, carry):
                j = _NBUF * i
                for b in range(_NBUF):
                    pltpu.async_copy(
                        bufs[0], acc.at[dst_v.at[j + b]], ssem[b], add=True)
                for b in range(_NBUF):
                    pltpu.make_async_copy(
                        bufs[0], acc.at[dst_v.at[j + b]], ssem[b]).wait()
                return carry

            lax.fori_loop(0, _NCHUNK // _NBUF, cbody, 0)
            finish(6 + c)

    return seg_sum


def _full(w):
    return pl.BlockSpec(w.shape, lambda i: (0,) * w.ndim)


def _node_mlp(x, geo_w, geo_b, n1_w, n1_b, n2_w, n2_b):
    def body(x_ref, gw, gb, w1, b1, w2, b2, h0, h1, h2):
        xg = x_ref[:, :128]
        xo = x_ref[:, 128:]
        geo = jnp.maximum(_dot(xg, gw[...]) + gb[...], 0.0)
        hp = jnp.maximum(
            _dot(geo, w1[:16, :]) + _dot(xo, w1[16:, :]) + b1[...], 0.0)
        h = _dot(hp, w2[...]) + b2[...]
        h0[...] = h[:, :_G]
        h1[...] = h[:, _G:2 * _G]
        h2[...] = h[:, 2 * _G:]

    gspec = pl.BlockSpec((_BN, _G), lambda i: (i, 0))
    gshape = jax.ShapeDtypeStruct((_N, _G), _F32)
    return pl.pallas_call(
        body,
        grid=(_N // _BN,),
        in_specs=[
            pl.BlockSpec((_BN, 256), lambda i: (i, 0)),
            _full(geo_w), _full(geo_b), _full(n1_w), _full(n1_b),
            _full(n2_w), _full(n2_b),
        ],
        out_specs=[gspec, gspec, gspec],
        out_shape=[gshape, gshape, gshape],
    )(x, geo_w, geo_b, n1_w, n1_b, n2_w, n2_b)


def _edge_tables(edge_attr, e1_w, e1_b):
    def body(a_ref, w, b, t0, t1, t2):
        r = jnp.maximum(_dot(a_ref[...], w[...]) + b[...], 0.0)
        t0[...] = r[:, :_G]
        t1[...] = r[:, _G:2 * _G]
        t2[...] = r[:, 2 * _G:]

    gspec = pl.BlockSpec((_BE, _G), lambda i: (i, 0))
    gshape = jax.ShapeDtypeStruct((_EPAD, _G), _F32)
    return pl.pallas_call(
        body,
        grid=(_EPAD // _BE,),
        in_specs=[
            pl.BlockSpec((_BE, 16), lambda i: (i, 0)),
            _full(e1_w), _full(e1_b),
        ],
        out_specs=[gspec, gspec, gspec],
        out_shape=[gshape, gshape, gshape],
    )(edge_attr, e1_w, e1_b)


def _sspec(g):
    return pl.BlockSpec(
        (1, _BN, _G), lambda i, g=g: (2 * g + i // 13, i - 13 * (i // 13), 0))


def _ce_combine(se, e2_w, e2_b):
    def body(s0r, s1r, s2r, ctr, w, b, ce):
        a = (_dot(s0r[0], w[:_G, :]) + _dot(s1r[0], w[_G:2 * _G, :])
             + _dot(s2r[0], w[2 * _G:, :]))
        ce[...] = a + ctr[0][:, :1] * b[...]

    return pl.pallas_call(
        body,
        grid=(_N // _BN,),
        in_specs=[_sspec(0), _sspec(1), _sspec(2), _sspec(3),
                  _full(e2_w), _full(e2_b)],
        out_specs=pl.BlockSpec((_BN, _H), lambda i: (i, 0)),
        out_shape=jax.ShapeDtypeStruct((_N, _H), _F32),
    )(se, se, se, se, e2_w, e2_b)


def _layer(h0, h1, h2, sh, ce, wx, we, u1, ub1, u2, ub2, lg, lb):
    def body(h0r, h1r, h2r, s0r, s1r, s2r, cer, wxr, wer, u1r, ub1r,
             u2r, ub2r, lgr, lbr, o0, o1, o2, osum):
        i = pl.program_id(0)
        agg = (_dot(s0r[0], wxr[:_G, :]) + _dot(s1r[0], wxr[_G:2 * _G, :])
               + _dot(s2r[0], wxr[2 * _G:, :])
               + _dot(cer[...], wer[...]))
        t = jnp.maximum(_dot(agg, u1r[...]) + ub1r[...], 0.0)
        upd = _dot(t, u2r[...]) + ub2r[...]
        r = jnp.concatenate([h0r[...], h1r[...], h2r[...]], axis=1) + upd
        m = jnp.mean(r, axis=1, keepdims=True)
        d = r - m
        v = jnp.mean(d * d, axis=1, keepdims=True)
        hn = d * lax.rsqrt(v + 1e-5) * lgr[...] + lbr[...]
        o0[...] = hn[:, :_G]
        o1[...] = hn[:, _G:2 * _G]
        o2[...] = hn[:, 2 * _G:]

        @pl.when(i == 0)
        def _():
            osum[...] = jnp.zeros((1, _H), _F32)

        osum[...] += jnp.sum(hn, axis=0, keepdims=True)

    gspec = pl.BlockSpec((_BN, _G), lambda i: (i, 0))
    gshape = jax.ShapeDtypeStruct((_N, _G), _F32)
    return pl.pallas_call(
        body,
        grid=(_N // _BN,),
        in_specs=[
            gspec, gspec, gspec, _sspec(0), _sspec(1), _sspec(2),
            pl.BlockSpec((_BN, _H), lambda i: (i, 0)),
            _full(wx), _full(we), _full(u1), _full(ub1), _full(u2),
            _full(ub2), _full(lg), _full(lb),
        ],
        out_specs=[gspec, gspec, gspec,
                   pl.BlockSpec((1, _H), lambda i: (0, 0))],
        out_shape=[gshape, gshape, gshape,
                   jax.ShapeDtypeStruct((1, _H), _F32)],
    )(h0, h1, h2, sh, sh, sh, ce, wx, we, u1, ub1, u2, ub2, lg, lb)


def _head(hsum, h1_w, h1_b, h2_w, h2_b):
    def body(sr, w1, b1, w2, b2, o):
        g = sr[...] * (1.0 / _N)
        t = jnp.maximum(_dot(g, w1[...]) + b1[...], 0.0)
        o[...] = _dot(t, w2[...]) + b2[...]

    return pl.pallas_call(
        body,
        grid=(1,),
        in_specs=[_full(hsum), _full(h1_w), _full(h1_b), _full(h2_w),
                  _full(h2_b)],
        out_specs=pl.BlockSpec((1, 128), lambda i: (0, 0)),
        out_shape=jax.ShapeDtypeStruct((1, 128), _F32),
    )(hsum, h1_w, h1_b, h2_w, h2_b)


def kernel(x, edge_index, edge_attr, geo_w, geo_b, n1_w, n1_b, n2_w, n2_b,
           e1_w, e1_b, e2_w, e2_b, msgx_w, msge_w, upd1_w, upd1_b, upd2_w,
           upd2_b, ln_g, ln_b, h1_w, h1_b, h2_w, h2_b):
    src = edge_index[0]
    dst = edge_index[1]
    pad = _EPAD - _E
    srcp = jnp.concatenate(
        [src, jnp.zeros((pad,), jnp.int32)]).reshape(_NS, _NCHUNK, _CH)
    dstp = jnp.concatenate(
        [dst, jnp.full((pad,), _N, jnp.int32)]).reshape(_NS, _NCHUNK, _CH)
    eap = jnp.concatenate(
        [edge_attr, jnp.zeros((pad, 16), _F32)], axis=0)
    zacc = jnp.zeros((_NACCH, _G), _F32)
    ones_row = jnp.zeros((_CH, _G), _F32).at[:, 0].set(1.0)

    r2 = lambda b: b.reshape(1, -1)

    h0, h1, h2 = _node_mlp(x, geo_w, r2(geo_b), n1_w, r2(n1_b),
                           n2_w, r2(n2_b))
    t0, t1, t2 = _edge_tables(eap, e1_w, r2(e1_b))
    se = _make_seg_sum(_EPAD, True, True)(t0, t1, t2, srcp, dstp, zacc,
                                          ones_row)
    ce = _ce_combine(se, e2_w, r2(e2_b))

    hsum = None
    for l in range(3):
        sh = _make_seg_sum(_N, False, False)(h0, h1, h2, srcp, dstp, zacc,
                                             ones_row)
        h0, h1, h2, hsum = _layer(h0, h1, h2, sh, ce,
                                  msgx_w[l], msge_w[l],
                                  upd1_w[l], r2(upd1_b[l]),
                                  upd2_w[l], r2(upd2_b[l]),
                                  r2(ln_g[l]), r2(ln_b[l]))

    return _head(hsum, h1_w, r2(h1_b), h2_w, r2(h2_b))
